# Initial kernel scaffold; baseline (speedup 1.0000x reference)
import jax
import jax.numpy as jnp
from jax.experimental import pallas as pl


def kernel(input, weights):
    raise NotImplementedError("write your pallas kernel here")


# speedup vs baseline: 2.0073x; 2.0073x over previous
import jax
import jax.numpy as jnp
from jax import lax
from jax.experimental import pallas as pl
from jax.experimental.pallas import tpu as pltpu
from jax.experimental.pallas import tpu_sc as plsc

EMB = 1024
PAD = 1
L = 16
NC, NS = 2, 16
NW = NC * NS
BSZ, SEQ = 4, 4096
ROWS = BSZ * SEQ
RPW = ROWS // NW
WPB = NW // BSZ
CPW = SEQ // WPB
CH = 32
NG = RPW // CH


def _body(tok_hbm, w_hbm, out_hbm, tok_v, idx_v, buf0, buf1,
          gs0, gs1, ws0, ws1):
    wid = lax.axis_index("s") * NC + lax.axis_index("c")
    b = wid // WPB
    c = wid % WPB
    pltpu.sync_copy(tok_hbm.at[b], tok_v)

    def pre(i, acc):
        grp = tok_v[pl.ds(i * L, L)]
        return acc + jnp.where(grp == PAD, 0, 1)

    acc = lax.fori_loop(0, c * (CPW // L), pre, jnp.zeros((L,), jnp.int32))
    carry0 = jnp.sum(acc)

    def pos_step(j, carry):
        grp = tok_v[pl.ds(c * CPW + j * L, L)]
        m = jnp.where(grp == PAD, 0, 1)
        cs = plsc.cumsum(m)
        idx_v[pl.ds(j * L, L)] = (carry + cs) * m + PAD
        return carry + jnp.sum(m)

    lax.fori_loop(0, CPW // L, pos_step, carry0)

    base = wid * RPW
    bufs = (buf0, buf1)
    gsems = (gs0, gs1)
    wsems = (ws0, ws1)

    def start(g):
        return pltpu.async_copy(w_hbm.at[idx_v.at[pl.ds(g * CH, CH)]],
                                bufs[g % 2], gsems[g % 2])

    gh = [None] * NG
    wh = [None] * NG
    gh[0] = start(0)
    for g in range(NG):
        p = g % 2
        if g + 1 < NG:
            if g - 1 >= 0:
                wh[g - 1].wait()
            gh[g + 1] = start(g + 1)
        gh[g].wait()
        wh[g] = pltpu.async_copy(bufs[p],
                                 out_hbm.at[pl.ds(base + g * CH, CH)],
                                 wsems[p])
    wh[NG - 2].wait()
    wh[NG - 1].wait()


@jax.jit
def _sc_embed(tokens, weights):
    mesh = plsc.VectorSubcoreMesh(core_axis_name="c", subcore_axis_name="s",
                                  num_cores=NC, num_subcores=NS)
    return pl.kernel(
        _body,
        out_type=jax.ShapeDtypeStruct((ROWS, EMB), jnp.float32),
        mesh=mesh,
        compiler_params=pltpu.CompilerParams(needs_layout_passes=False),
        scratch_types=[
            pltpu.VMEM((SEQ,), jnp.int32),
            pltpu.VMEM((RPW,), jnp.int32),
            pltpu.VMEM((CH, EMB), jnp.float32),
            pltpu.VMEM((CH, EMB), jnp.float32),
            pltpu.SemaphoreType.DMA,
            pltpu.SemaphoreType.DMA,
            pltpu.SemaphoreType.DMA,
            pltpu.SemaphoreType.DMA,
        ],
    )(tokens, weights)


def kernel(input, weights):
    bsz, seq_len = input.shape
    out = _sc_embed(input, weights)
    return lax.stop_gradient(out.reshape(bsz, seq_len, -1))


# speedup vs baseline: 2.0079x; 1.0003x over previous
import jax
import jax.numpy as jnp
from jax import lax
from jax.experimental import pallas as pl
from jax.experimental.pallas import tpu as pltpu
from jax.experimental.pallas import tpu_sc as plsc

EMB = 1024
PAD = 1
L = 16
NC, NS = 2, 16
NW = NC * NS
BSZ, SEQ = 4, 4096
ROWS = BSZ * SEQ
RPW = ROWS // NW
WPB = NW // BSZ
CPW = SEQ // WPB
CH = 32
NG = RPW // CH


def _body(tok_hbm, w_hbm, out_hbm, tok_v, idx_v, buf0, buf1, buf2,
          gs0, gs1, gs2, ws0, ws1, ws2):
    wid = lax.axis_index("s") * NC + lax.axis_index("c")
    b = wid // WPB
    c = wid % WPB
    pltpu.sync_copy(tok_hbm.at[b], tok_v)

    def pre(i, acc):
        grp = tok_v[pl.ds(i * L, L)]
        return acc + jnp.where(grp == PAD, 0, 1)

    acc = lax.fori_loop(0, c * (CPW // L), pre, jnp.zeros((L,), jnp.int32))
    carry0 = jnp.sum(acc)

    def pos_step(j, carry):
        grp = tok_v[pl.ds(c * CPW + j * L, L)]
        m = jnp.where(grp == PAD, 0, 1)
        cs = plsc.cumsum(m)
        idx_v[pl.ds(j * L, L)] = (carry + cs) * m + PAD
        return carry + jnp.sum(m)

    lax.fori_loop(0, CPW // L, pos_step, carry0)

    base = wid * RPW
    bufs = (buf0, buf1, buf2)
    gsems = (gs0, gs1, gs2)
    wsems = (ws0, ws1, ws2)
    NB = 3

    def start(g):
        return pltpu.async_copy(w_hbm.at[idx_v.at[pl.ds(g * CH, CH)]],
                                bufs[g % NB], gsems[g % NB])

    gh = [None] * NG
    wh = [None] * NG
    gh[0] = start(0)
    gh[1] = start(1)
    for g in range(NG):
        p = g % NB
        if g + 2 < NG:
            if g - 1 >= 0:
                wh[g - 1].wait()
            gh[g + 2] = start(g + 2)
        gh[g].wait()
        wh[g] = pltpu.async_copy(bufs[p],
                                 out_hbm.at[pl.ds(base + g * CH, CH)],
                                 wsems[p])
    wh[NG - 2].wait()
    wh[NG - 1].wait()


@jax.jit
def _sc_embed(tokens, weights):
    mesh = plsc.VectorSubcoreMesh(core_axis_name="c", subcore_axis_name="s",
                                  num_cores=NC, num_subcores=NS)
    return pl.kernel(
        _body,
        out_type=jax.ShapeDtypeStruct((ROWS, EMB), jnp.float32),
        mesh=mesh,
        compiler_params=pltpu.CompilerParams(needs_layout_passes=False,
                                             skip_device_barrier=True),
        scratch_types=[
            pltpu.VMEM((SEQ,), jnp.int32),
            pltpu.VMEM((RPW,), jnp.int32),
            pltpu.VMEM((CH, EMB), jnp.float32),
            pltpu.VMEM((CH, EMB), jnp.float32),
            pltpu.VMEM((CH, EMB), jnp.float32),
            pltpu.SemaphoreType.DMA,
            pltpu.SemaphoreType.DMA,
            pltpu.SemaphoreType.DMA,
            pltpu.SemaphoreType.DMA,
            pltpu.SemaphoreType.DMA,
            pltpu.SemaphoreType.DMA,
        ],
    )(tokens, weights)


def kernel(input, weights):
    bsz, seq_len = input.shape
    out = _sc_embed(input, weights)
    return lax.stop_gradient(out.reshape(bsz, seq_len, -1))
